# Initial kernel scaffold; baseline (speedup 1.0000x reference)
import jax
import jax.numpy as jnp
from jax.experimental import pallas as pl


def kernel(x, w1, b1, w2, b2, w3, b3, w4, b4, fc1_w, fc1_b, clf_w, clf_b):
    raise NotImplementedError("write your pallas kernel here")


# speedup vs baseline: 2.4986x; 2.4986x over previous
import jax
import jax.numpy as jnp
from jax.experimental import pallas as pl
from jax.experimental.pallas import tpu as pltpu

_B = 16


def _conv_out(h):
    return (h - 1) // 2 + 1


def _split_bf16(a):
    hi = a.astype(jnp.bfloat16)
    lo = (a - hi.astype(jnp.float32)).astype(jnp.bfloat16)
    return hi, lo


def _windows(x_pad, ho, wo, cin):
    b = x_pad.shape[0]
    hq, wq, wp = ho + 1, wo + 1, 2 * wo + 2
    xr = x_pad.reshape(b, hq, 2, wp, cin)
    out = []
    for dy in range(3):
        rows = xr[:, dy // 2: dy // 2 + ho, dy % 2]
        cols = rows.reshape(b, ho, wq, 2, cin)
        for dx in range(3):
            win = cols[:, :, dx // 2: dx // 2 + wo, dx % 2: dx % 2 + 1, :]
            out.append(win.reshape(b * ho * wo, cin))
    return out


def _dot2(ic_ref, wh_ref, wl_ref, b_ref):
    ic = ic_ref[...]
    return (jnp.dot(ic, wh_ref[...], preferred_element_type=jnp.float32)
            + jnp.dot(ic, wl_ref[...], preferred_element_type=jnp.float32)
            + b_ref[...])


def _make_conv_kernel(ho1, wo1, ho2, wo2, ho3, wo3, ho4, wo4):
    def body(x_ref, w1h_ref, w1l_ref, b1_ref, w2h_ref, w2l_ref, b2_ref,
             w3h_ref, w3l_ref, b3_ref, w4h_ref, w4l_ref, b4_ref,
             o32_ref, o16_ref, ic2, ic3, ic4, xp2, xp3, xp4):
        b = _B

        x = x_ref[...].reshape(b * ho1 * wo1, 48)
        y1 = jnp.maximum(
            jnp.dot(x, w1h_ref[...], preferred_element_type=jnp.float32)
            + jnp.dot(x, w1l_ref[...], preferred_element_type=jnp.float32)
            + b1_ref[...], 0.0).astype(jnp.bfloat16)

        xp2[...] = jnp.zeros_like(xp2)
        xp2[:, 1:1 + ho1, 1:1 + wo1, :] = y1.reshape(b, ho1, wo1, 32)
        for t, win in enumerate(_windows(xp2[...], ho2, wo2, 32)):
            ic2[:, t * 32:(t + 1) * 32] = win
        y2 = jnp.maximum(_dot2(ic2, w2h_ref, w2l_ref, b2_ref),
                         0.0).astype(jnp.bfloat16)

        xp3[...] = jnp.zeros_like(xp3)
        xp3[:, 1:1 + ho2, 1:1 + wo2, :] = y2.reshape(b, ho2, wo2, 64)
        for t, win in enumerate(_windows(xp3[...], ho3, wo3, 64)):
            ic3[:, t * 64:(t + 1) * 64] = win
        y3 = jnp.maximum(_dot2(ic3, w3h_ref, w3l_ref, b3_ref),
                         0.0).astype(jnp.bfloat16)

        xp4[...] = jnp.zeros_like(xp4)
        xp4[:, 1:1 + ho3, 1:1 + wo3, :] = y3.reshape(b, ho3, wo3, 128)
        for t, win in enumerate(_windows(xp4[...], ho4, wo4, 128)):
            ic4[:, t * 128:(t + 1) * 128] = win
        y4 = jnp.maximum(_dot2(ic4, w4h_ref, w4l_ref, b4_ref),
                         0.0)

        o32_ref[...] = y4.reshape(b, ho4 * wo4, 256)
        o16_ref[...] = y4.astype(jnp.bfloat16).reshape(b, ho4 * wo4, 256)

    return body


def _conv_stack(xs, wbs, ho1, wo1, ho2, wo2, ho3, wo3, ho4, wo4):
    n = xs.shape[0]
    b = _B
    hw4 = ho4 * wo4
    body = _make_conv_kernel(ho1, wo1, ho2, wo2, ho3, wo3, ho4, wo4)

    def _w_spec(a):
        return pl.BlockSpec(a.shape, lambda i: (0,) * a.ndim)

    return pl.pallas_call(
        body,
        out_shape=[jax.ShapeDtypeStruct((n, hw4, 256), jnp.float32),
                   jax.ShapeDtypeStruct((n, hw4, 256), jnp.bfloat16)],
        grid=(n // b,),
        in_specs=([pl.BlockSpec((b, ho1 * wo1, 48), lambda i: (i, 0, 0))]
                  + [_w_spec(a) for a in wbs]),
        out_specs=[pl.BlockSpec((b, hw4, 256), lambda i: (i, 0, 0)),
                   pl.BlockSpec((b, hw4, 256), lambda i: (i, 0, 0))],
        scratch_shapes=[
            pltpu.VMEM((b * ho2 * wo2, 288), jnp.bfloat16),
            pltpu.VMEM((b * ho3 * wo3, 576), jnp.bfloat16),
            pltpu.VMEM((b * ho4 * wo4, 1152), jnp.bfloat16),
            pltpu.VMEM((b, 2 * ho2 + 2, 2 * wo2 + 2, 32), jnp.bfloat16),
            pltpu.VMEM((b, 2 * ho3 + 2, 2 * wo3 + 2, 64), jnp.bfloat16),
            pltpu.VMEM((b, 2 * ho4 + 2, 2 * wo4 + 2, 128), jnp.bfloat16),
        ],
        compiler_params=pltpu.CompilerParams(
            dimension_semantics=("parallel",)),
    )(xs, *wbs)


def _fc_body(f_ref, w1h_ref, w1l_ref, b1_ref, w2h_ref, w2l_ref, b2_ref,
             o_ref):
    f = f_ref[...]
    h = jnp.maximum(
        jnp.dot(f, w1h_ref[...], preferred_element_type=jnp.float32)
        + jnp.dot(f, w1l_ref[...], preferred_element_type=jnp.float32)
        + b1_ref[...], 0.0).astype(jnp.bfloat16)
    o_ref[...] = (jnp.dot(h, w2h_ref[...], preferred_element_type=jnp.float32)
                  + jnp.dot(h, w2l_ref[...], preferred_element_type=jnp.float32)
                  + b2_ref[...])


def _fc_head(feat16, fc1_w, fc1_b, clf_w, clf_b):
    n, k = feat16.shape
    d, c = clf_w.shape
    bm = n // 2
    w1h, w1l = _split_bf16(fc1_w)
    w2h, w2l = _split_bf16(clf_w)
    return pl.pallas_call(
        _fc_body,
        out_shape=jax.ShapeDtypeStruct((n, c), jnp.float32),
        grid=(2,),
        in_specs=[
            pl.BlockSpec((bm, k), lambda i: (i, 0)),
            pl.BlockSpec((k, d), lambda i: (0, 0)),
            pl.BlockSpec((k, d), lambda i: (0, 0)),
            pl.BlockSpec((1, d), lambda i: (0, 0)),
            pl.BlockSpec((d, c), lambda i: (0, 0)),
            pl.BlockSpec((d, c), lambda i: (0, 0)),
            pl.BlockSpec((1, c), lambda i: (0, 0)),
        ],
        out_specs=pl.BlockSpec((bm, c), lambda i: (i, 0)),
        compiler_params=pltpu.CompilerParams(
            dimension_semantics=("parallel",)),
    )(feat16, w1h, w1l, fc1_b, w2h, w2l, clf_b)


def kernel(x, w1, b1, w2, b2, w3, b3, w4, b4, fc1_w, fc1_b, clf_w, clf_b):
    n, _, h0, w0 = x.shape
    ho1, wo1 = _conv_out(h0), _conv_out(w0)
    ho2, wo2 = _conv_out(ho1), _conv_out(wo1)
    ho3, wo3 = _conv_out(ho2), _conv_out(wo2)
    ho4, wo4 = _conv_out(ho3), _conv_out(wo3)

    x_nhwc = jnp.transpose(x, (0, 2, 3, 1))
    xp = jnp.pad(x_nhwc, ((0, 0), (1, 1), (1, 1), (0, 0)))
    xs = (xp.reshape(n, ho1 + 1, 2, wo1 + 1, 2, 3)
          .transpose(0, 1, 3, 2, 4, 5)
          .reshape(n, ho1 + 1, wo1 + 1, 12))
    xw = jnp.concatenate(
        [xs[:, r:r + ho1, c:c + wo1, :] for r in range(2) for c in range(2)],
        axis=-1).reshape(n, ho1 * wo1, 48).astype(jnp.bfloat16)

    w1e = jnp.zeros((2, 2, 12, 32), jnp.float32)
    for dy in range(3):
        for dx in range(3):
            lane = 6 * (dy % 2) + 3 * (dx % 2)
            w1e = w1e.at[dy // 2, dx // 2, lane:lane + 3, :].set(w1[dy, dx])

    w1h, w1l = _split_bf16(w1e.reshape(48, 32))
    w2h, w2l = _split_bf16(w2.reshape(9 * 32, 64))
    w3h, w3l = _split_bf16(w3.reshape(9 * 64, 128))
    w4h, w4l = _split_bf16(w4.reshape(9 * 128, 256))
    wbs = (w1h, w1l, b1, w2h, w2l, b2, w3h, w3l, b3, w4h, w4l, b4)

    feat32, feat16 = _conv_stack(xw, wbs, ho1, wo1, ho2, wo2, ho3, wo3,
                                 ho4, wo4)
    logits = _fc_head(feat16.reshape(n, ho4 * wo4 * 256),
                      fc1_w, fc1_b, clf_w, clf_b)
    feat_pt = jnp.transpose(feat32, (0, 2, 1)).reshape(n, -1)
    return feat_pt, logits


# speedup vs baseline: 2.5785x; 1.0320x over previous
import jax
import jax.numpy as jnp
from jax.experimental import pallas as pl
from jax.experimental.pallas import tpu as pltpu

_B = 16


def _conv_out(h):
    return (h - 1) // 2 + 1


def _split_bf16(a):
    hi = a.astype(jnp.bfloat16)
    lo = (a - hi.astype(jnp.float32)).astype(jnp.bfloat16)
    return hi, lo


def _windows(x_pad, ho, wo, cin):
    b = x_pad.shape[0]
    hq, wq, wp = ho + 1, wo + 1, 2 * wo + 2
    xr = x_pad.reshape(b, hq, 2, wp, cin)
    out = []
    for dy in range(3):
        rows = xr[:, dy // 2: dy // 2 + ho, dy % 2]
        cols = rows.reshape(b, ho, wq, 2, cin)
        for dx in range(3):
            win = cols[:, :, dx // 2: dx // 2 + wo, dx % 2: dx % 2 + 1, :]
            out.append(win.reshape(b * ho * wo, cin))
    return out


def _dot_ns(ic_ref, ws_ref, b_ref, cout):
    out = jnp.dot(ic_ref[...], ws_ref[...],
                  preferred_element_type=jnp.float32)
    return out[:, :cout] + out[:, cout:] + b_ref[...]


def _make_conv_kernel(ho1, wo1, ho2, wo2, ho3, wo3, ho4, wo4):
    def body(x_ref, w1_ref, b1_ref, w2_ref, b2_ref,
             w3_ref, b3_ref, w4_ref, b4_ref,
             o32_ref, o16_ref, ic2, ic3, ic4, xp2, xp3, xp4):
        b = _B

        x = x_ref[...].reshape(b * ho1 * wo1, 96)
        y1 = jnp.maximum(
            jnp.dot(x, w1_ref[...], preferred_element_type=jnp.float32)
            + b1_ref[...], 0.0).astype(jnp.bfloat16)

        xp2[...] = jnp.zeros_like(xp2)
        xp2[:, 1:1 + ho1, 1:1 + wo1, :] = y1.reshape(b, ho1, wo1, 32)
        for t, win in enumerate(_windows(xp2[...], ho2, wo2, 32)):
            ic2[:, t * 32:(t + 1) * 32] = win
        y2 = jnp.maximum(_dot_ns(ic2, w2_ref, b2_ref, 64),
                         0.0).astype(jnp.bfloat16)

        xp3[...] = jnp.zeros_like(xp3)
        xp3[:, 1:1 + ho2, 1:1 + wo2, :] = y2.reshape(b, ho2, wo2, 64)
        for t, win in enumerate(_windows(xp3[...], ho3, wo3, 64)):
            ic3[:, t * 64:(t + 1) * 64] = win
        y3 = jnp.maximum(_dot_ns(ic3, w3_ref, b3_ref, 128),
                         0.0).astype(jnp.bfloat16)

        xp4[...] = jnp.zeros_like(xp4)
        xp4[:, 1:1 + ho3, 1:1 + wo3, :] = y3.reshape(b, ho3, wo3, 128)
        for t, win in enumerate(_windows(xp4[...], ho4, wo4, 128)):
            ic4[:, t * 128:(t + 1) * 128] = win
        y4 = jnp.maximum(_dot_ns(ic4, w4_ref, b4_ref, 256),
                         0.0)

        o32_ref[...] = y4.reshape(b, ho4 * wo4, 256)
        o16_ref[...] = y4.astype(jnp.bfloat16).reshape(b, ho4 * wo4, 256)

    return body


def _conv_stack(xs, wbs, ho1, wo1, ho2, wo2, ho3, wo3, ho4, wo4):
    n = xs.shape[0]
    b = _B
    hw4 = ho4 * wo4
    body = _make_conv_kernel(ho1, wo1, ho2, wo2, ho3, wo3, ho4, wo4)

    def _w_spec(a):
        return pl.BlockSpec(a.shape, lambda i: (0,) * a.ndim)

    return pl.pallas_call(
        body,
        out_shape=[jax.ShapeDtypeStruct((n, hw4, 256), jnp.float32),
                   jax.ShapeDtypeStruct((n, hw4, 256), jnp.bfloat16)],
        grid=(n // b,),
        in_specs=([pl.BlockSpec((b, ho1 * wo1, 96), lambda i: (i, 0, 0))]
                  + [_w_spec(a) for a in wbs]),
        out_specs=[pl.BlockSpec((b, hw4, 256), lambda i: (i, 0, 0)),
                   pl.BlockSpec((b, hw4, 256), lambda i: (i, 0, 0))],
        scratch_shapes=[
            pltpu.VMEM((b * ho2 * wo2, 288), jnp.bfloat16),
            pltpu.VMEM((b * ho3 * wo3, 576), jnp.bfloat16),
            pltpu.VMEM((b * ho4 * wo4, 1152), jnp.bfloat16),
            pltpu.VMEM((b, 2 * ho2 + 2, 2 * wo2 + 2, 32), jnp.bfloat16),
            pltpu.VMEM((b, 2 * ho3 + 2, 2 * wo3 + 2, 64), jnp.bfloat16),
            pltpu.VMEM((b, 2 * ho4 + 2, 2 * wo4 + 2, 128), jnp.bfloat16),
        ],
        compiler_params=pltpu.CompilerParams(
            dimension_semantics=("parallel",)),
    )(xs, *wbs)


def _fc_body(f_ref, w1h_ref, w1l_ref, b1_ref, w2h_ref, w2l_ref, b2_ref,
             o_ref):
    f = f_ref[...]
    h = jnp.maximum(
        jnp.dot(f, w1h_ref[...], preferred_element_type=jnp.float32)
        + jnp.dot(f, w1l_ref[...], preferred_element_type=jnp.float32)
        + b1_ref[...], 0.0).astype(jnp.bfloat16)
    o_ref[...] = (jnp.dot(h, w2h_ref[...], preferred_element_type=jnp.float32)
                  + jnp.dot(h, w2l_ref[...], preferred_element_type=jnp.float32)
                  + b2_ref[...])


def _fc_head(feat16, fc1_w, fc1_b, clf_w, clf_b):
    n, k = feat16.shape
    d, c = clf_w.shape
    bm = n // 2
    w1h, w1l = _split_bf16(fc1_w)
    w2h, w2l = _split_bf16(clf_w)
    return pl.pallas_call(
        _fc_body,
        out_shape=jax.ShapeDtypeStruct((n, c), jnp.float32),
        grid=(2,),
        in_specs=[
            pl.BlockSpec((bm, k), lambda i: (i, 0)),
            pl.BlockSpec((k, d), lambda i: (0, 0)),
            pl.BlockSpec((k, d), lambda i: (0, 0)),
            pl.BlockSpec((1, d), lambda i: (0, 0)),
            pl.BlockSpec((d, c), lambda i: (0, 0)),
            pl.BlockSpec((d, c), lambda i: (0, 0)),
            pl.BlockSpec((1, c), lambda i: (0, 0)),
        ],
        out_specs=pl.BlockSpec((bm, c), lambda i: (i, 0)),
        compiler_params=pltpu.CompilerParams(
            dimension_semantics=("parallel",)),
    )(feat16, w1h, w1l, fc1_b, w2h, w2l, clf_b)


def kernel(x, w1, b1, w2, b2, w3, b3, w4, b4, fc1_w, fc1_b, clf_w, clf_b):
    n, _, h0, w0 = x.shape
    ho1, wo1 = _conv_out(h0), _conv_out(w0)
    ho2, wo2 = _conv_out(ho1), _conv_out(wo1)
    ho3, wo3 = _conv_out(ho2), _conv_out(wo2)
    ho4, wo4 = _conv_out(ho3), _conv_out(wo3)

    x_nhwc = jnp.transpose(x, (0, 2, 3, 1))
    xp = jnp.pad(x_nhwc, ((0, 0), (1, 1), (1, 1), (0, 0)))
    xs = (xp.reshape(n, ho1 + 1, 2, wo1 + 1, 2, 3)
          .transpose(0, 1, 3, 2, 4, 5)
          .reshape(n, ho1 + 1, wo1 + 1, 12))
    wins = [xs[:, r:r + ho1, c:c + wo1, :] for r in range(2) for c in range(2)]
    xw = (jnp.concatenate(wins + wins, axis=-1)
          .reshape(n, ho1 * wo1, 96).astype(jnp.bfloat16))

    w1e = jnp.zeros((2, 2, 12, 32), jnp.float32)
    for dy in range(3):
        for dx in range(3):
            lane = 6 * (dy % 2) + 3 * (dx % 2)
            w1e = w1e.at[dy // 2, dx // 2, lane:lane + 3, :].set(w1[dy, dx])

    w1h, w1l = _split_bf16(w1e.reshape(48, 32))
    w2h, w2l = _split_bf16(w2.reshape(9 * 32, 64))
    w3h, w3l = _split_bf16(w3.reshape(9 * 64, 128))
    w4h, w4l = _split_bf16(w4.reshape(9 * 128, 256))
    w1k = jnp.concatenate([w1h, w1l], axis=0)
    w2s = jnp.concatenate([w2h, w2l], axis=1)
    w3s = jnp.concatenate([w3h, w3l], axis=1)
    w4s = jnp.concatenate([w4h, w4l], axis=1)
    wbs = (w1k, b1, w2s, b2, w3s, b3, w4s, b4)

    feat32, feat16 = _conv_stack(xw, wbs, ho1, wo1, ho2, wo2, ho3, wo3,
                                 ho4, wo4)
    logits = _fc_head(feat16.reshape(n, ho4 * wo4 * 256),
                      fc1_w, fc1_b, clf_w, clf_b)
    feat_pt = jnp.transpose(feat32, (0, 2, 1)).reshape(n, -1)
    return feat_pt, logits


# speedup vs baseline: 5.1094x; 1.9815x over previous
import jax
import jax.numpy as jnp
from jax.experimental import pallas as pl
from jax.experimental.pallas import tpu as pltpu

_B = 16


def _conv_out(h):
    return (h - 1) // 2 + 1


def _split_bf16(a):
    hi = a.astype(jnp.bfloat16)
    lo = (a - hi.astype(jnp.float32)).astype(jnp.bfloat16)
    return hi, lo


def _store_parity(y, ys_ref, xm_ref, th, tw):
    b, hc, wc, c = y.shape
    xm_ref[...] = jnp.zeros_like(xm_ref)
    ys_ref[...] = y
    ne = (th + 1) // 2
    no = th // 2
    nce = (tw + 1) // 2
    nco = tw // 2
    for rp, nr in ((0, ne), (1, no)):
        for cp, nc in ((0, nce), (1, nco)):
            piece = ys_ref[:, pl.ds(rp, nr, 2), pl.ds(cp, nc, 2), :]
            xm_ref[:, 1:1 + nr, rp, 1:1 + nc, cp * c:(cp + 1) * c] = (
                piece.astype(jnp.bfloat16))


def _store_pairform(ym, xm_ref, th, tw):
    b, hc, g, c2 = ym.shape
    c = c2 // 2
    xm_ref[...] = jnp.zeros_like(xm_ref)
    yr = ym.reshape(b, hc // 2, 2, g, c2)
    ne = (th + 1) // 2
    no = th // 2
    nce = (tw + 1) // 2
    nco = tw // 2
    for rp, nr in ((0, ne), (1, no)):
        piece = yr[:, :, rp]
        xm_ref[:, 1:1 + nr, rp, 1:1 + nce, 0:c] = (
            piece[:, 0:nr, 0:nce, 0:c].astype(jnp.bfloat16))
        xm_ref[:, 1:1 + nr, rp, 1:1 + nco, c:2 * c] = (
            piece[:, 0:nr, 0:nco, c:2 * c].astype(jnp.bfloat16))


_TAP = ((1, 0), (0, 1), (1, 1))


def _gather_ic(xm_ref, ic_ref, cin, ho_c, wo_c):
    for dy in range(3):
        rp, ai = _TAP[dy]
        for dx in range(3):
            cp, aj = _TAP[dx]
            win = xm_ref[:, ai:ai + ho_c, rp, aj:aj + wo_c,
                         cp * cin:(cp + 1) * cin]
            t = 3 * dy + dx
            ic_ref[:, :, :, t * cin:(t + 1) * cin] = win


def _dot_ns(ic, ws_ref, b_ref, cout):
    out = jnp.dot(ic, ws_ref[...], preferred_element_type=jnp.float32)
    return out[:, :cout] + out[:, cout:] + b_ref[...]


def _conv_body(x_ref, w1_ref, b1_ref, w2_ref, b2_ref, w3_ref, b3_ref,
               w4_ref, b4_ref, o16_ref,
               ic2, ic3, ic4, xm2, xm3, xm4, ys3, ys4):
    b = _B

    x = x_ref[...].reshape(b * 42 * 24, 192)
    y1 = jnp.maximum(
        jnp.dot(x, w1_ref[...], preferred_element_type=jnp.float32)
        + b1_ref[...], 0.0)
    _store_pairform(y1.reshape(b, 42, 24, 64), xm2, 42, 42)

    _gather_ic(xm2, ic2, 32, 22, 24)
    y2 = jnp.maximum(
        _dot_ns(ic2[...].reshape(b * 22 * 24, 288), w2_ref, b2_ref, 64),
        0.0)
    _store_parity(y2.reshape(b, 22, 24, 64), ys3, xm3, 21, 21)

    _gather_ic(xm3, ic3, 64, 12, 16)
    y3 = jnp.maximum(
        _dot_ns(ic3[...].reshape(b * 12 * 16, 576), w3_ref, b3_ref, 128),
        0.0)
    _store_parity(y3.reshape(b, 12, 16, 128), ys4, xm4, 11, 11)

    _gather_ic(xm4, ic4, 128, 6, 8)
    y4 = jnp.maximum(
        _dot_ns(ic4[...].reshape(b * 6 * 8, 1152), w4_ref, b4_ref, 256),
        0.0)
    y4t = y4.reshape(b, 6, 8, 256)[:, :, 0:6, :].reshape(b, 36, 256)
    o16_ref[...] = y4t.astype(jnp.bfloat16)


def _conv_stack(xw, wbs, n):
    b = _B

    def _w_spec(a):
        return pl.BlockSpec(a.shape, lambda i: (0,) * a.ndim)

    return pl.pallas_call(
        _conv_body,
        out_shape=jax.ShapeDtypeStruct((n, 36, 256), jnp.bfloat16),
        grid=(n // b,),
        in_specs=([pl.BlockSpec((b, 42 * 24, 192), lambda i: (i, 0, 0))]
                  + [_w_spec(a) for a in wbs]),
        out_specs=pl.BlockSpec((b, 36, 256), lambda i: (i, 0, 0)),
        scratch_shapes=[
            pltpu.VMEM((b, 22, 24, 288), jnp.bfloat16),
            pltpu.VMEM((b, 12, 16, 576), jnp.bfloat16),
            pltpu.VMEM((b, 6, 8, 1152), jnp.bfloat16),
            pltpu.VMEM((b, 23, 2, 25, 64), jnp.bfloat16),
            pltpu.VMEM((b, 13, 2, 17, 128), jnp.bfloat16),
            pltpu.VMEM((b, 7, 2, 9, 256), jnp.bfloat16),
            pltpu.VMEM((b, 22, 24, 64), jnp.float32),
            pltpu.VMEM((b, 12, 16, 128), jnp.float32),
        ],
        compiler_params=pltpu.CompilerParams(
            dimension_semantics=("parallel",)),
    )(xw, *wbs)


def _fc_body(f_ref, w1h_ref, w1l_ref, b1_ref, w2h_ref, w2l_ref, b2_ref,
             o_ref):
    f = f_ref[...]
    h = jnp.maximum(
        jnp.dot(f, w1h_ref[...], preferred_element_type=jnp.float32)
        + jnp.dot(f, w1l_ref[...], preferred_element_type=jnp.float32)
        + b1_ref[...], 0.0).astype(jnp.bfloat16)
    o_ref[...] = (jnp.dot(h, w2h_ref[...], preferred_element_type=jnp.float32)
                  + jnp.dot(h, w2l_ref[...], preferred_element_type=jnp.float32)
                  + b2_ref[...])


def _fc_head(feat16, fc1_w, fc1_b, clf_w, clf_b):
    n, k = feat16.shape
    d, c = clf_w.shape
    bm = n // 2
    w1h, w1l = _split_bf16(fc1_w)
    w2h, w2l = _split_bf16(clf_w)
    return pl.pallas_call(
        _fc_body,
        out_shape=jax.ShapeDtypeStruct((n, c), jnp.float32),
        grid=(2,),
        in_specs=[
            pl.BlockSpec((bm, k), lambda i: (i, 0)),
            pl.BlockSpec((k, d), lambda i: (0, 0)),
            pl.BlockSpec((k, d), lambda i: (0, 0)),
            pl.BlockSpec((1, d), lambda i: (0, 0)),
            pl.BlockSpec((d, c), lambda i: (0, 0)),
            pl.BlockSpec((d, c), lambda i: (0, 0)),
            pl.BlockSpec((1, c), lambda i: (0, 0)),
        ],
        out_specs=pl.BlockSpec((bm, c), lambda i: (i, 0)),
        compiler_params=pltpu.CompilerParams(
            dimension_semantics=("parallel",)),
    )(feat16, w1h, w1l, fc1_b, w2h, w2l, clf_b)


def kernel(x, w1, b1, w2, b2, w3, b3, w4, b4, fc1_w, fc1_b, clf_w, clf_b):
    n, _, h0, w0 = x.shape
    ho1, wo1 = _conv_out(h0), _conv_out(w0)

    x_nhwc = jnp.transpose(x, (0, 2, 3, 1))
    xp = jnp.pad(x_nhwc, ((0, 0), (1, 1), (1, 1), (0, 0)))
    xs = (xp.reshape(n, ho1 + 1, 2, wo1 + 1, 2, 3)
          .transpose(0, 1, 3, 2, 4, 5)
          .reshape(n, ho1 + 1, wo1 + 1, 12))
    xsp = jnp.pad(xs, ((0, 0), (0, 0), (0, 6), (0, 0)))
    wins = [xsp[:, r:r + 42, c:c + 48, :] for r in range(2) for c in range(2)]
    xw = jnp.concatenate(wins, axis=-1).reshape(n, 42, 24, 96)
    xw = (jnp.concatenate([xw, xw], axis=-1)
          .reshape(n, 42 * 24, 192).astype(jnp.bfloat16))

    w1e = jnp.zeros((2, 2, 12, 32), jnp.float32)
    for dy in range(3):
        for dx in range(3):
            lane = 6 * (dy % 2) + 3 * (dx % 2)
            w1e = w1e.at[dy // 2, dx // 2, lane:lane + 3, :].set(w1[dy, dx])

    w1h, w1l = _split_bf16(w1e.reshape(48, 32))
    w2h, w2l = _split_bf16(w2.reshape(9 * 32, 64))
    w3h, w3l = _split_bf16(w3.reshape(9 * 64, 128))
    w4h, w4l = _split_bf16(w4.reshape(9 * 128, 256))
    z = jnp.zeros((48, 32), jnp.bfloat16)
    w1p = jnp.block([[w1h, z], [z, w1h], [w1l, z], [z, w1l]])
    b1p = jnp.concatenate([b1, b1], axis=1)
    w2s = jnp.concatenate([w2h, w2l], axis=1)
    w3s = jnp.concatenate([w3h, w3l], axis=1)
    w4s = jnp.concatenate([w4h, w4l], axis=1)
    wbs = (w1p, b1p, w2s, b2, w3s, b3, w4s, b4)

    feat16 = _conv_stack(xw, wbs, n)
    logits = _fc_head(feat16.reshape(n, 36 * 256),
                      fc1_w, fc1_b, clf_w, clf_b)
    feat_pt = (jnp.transpose(feat16, (0, 2, 1))
               .astype(jnp.float32).reshape(n, -1))
    return feat_pt, logits


# speedup vs baseline: 5.3528x; 1.0476x over previous
import jax
import jax.numpy as jnp
from jax.experimental import pallas as pl
from jax.experimental.pallas import tpu as pltpu

_B = 16


def _conv_out(h):
    return (h - 1) // 2 + 1


def _split_bf16(a):
    hi = a.astype(jnp.bfloat16)
    lo = (a - hi.astype(jnp.float32)).astype(jnp.bfloat16)
    return hi, lo


def _store_parity(y, ys_ref, xm_ref, th, tw):
    b, hc, wc, c = y.shape
    xm_ref[...] = jnp.zeros_like(xm_ref)
    ys_ref[...] = y
    ne = (th + 1) // 2
    no = th // 2
    nce = (tw + 1) // 2
    nco = tw // 2
    for rp, nr in ((0, ne), (1, no)):
        for cp, nc in ((0, nce), (1, nco)):
            piece = ys_ref[:, pl.ds(rp, nr, 2), pl.ds(cp, nc, 2), :]
            xm_ref[:, 1:1 + nr, rp, 1:1 + nc, cp * c:(cp + 1) * c] = (
                piece.astype(jnp.bfloat16))


def _store_pairform(ym, xm_ref, th, tw):
    b, hc, g, c2 = ym.shape
    c = c2 // 2
    xm_ref[...] = jnp.zeros_like(xm_ref)
    yr = ym.reshape(b, hc // 2, 2, g, c2)
    ne = (th + 1) // 2
    no = th // 2
    nce = (tw + 1) // 2
    nco = tw // 2
    for rp, nr in ((0, ne), (1, no)):
        piece = yr[:, :, rp]
        xm_ref[:, 1:1 + nr, rp, 1:1 + nce, 0:c] = (
            piece[:, 0:nr, 0:nce, 0:c].astype(jnp.bfloat16))
        xm_ref[:, 1:1 + nr, rp, 1:1 + nco, c:2 * c] = (
            piece[:, 0:nr, 0:nco, c:2 * c].astype(jnp.bfloat16))


_TAP = ((1, 0), (0, 1), (1, 1))


def _gather_ic(xm_ref, ic_ref, cin, ho_c, wo_c):
    for dy in range(3):
        rp, ai = _TAP[dy]
        base = dy * 3 * cin
        ic_ref[:, :, :, base:base + cin] = (
            xm_ref[:, ai:ai + ho_c, rp, 0:wo_c, cin:2 * cin])
        ic_ref[:, :, :, base + cin:base + 3 * cin] = (
            xm_ref[:, ai:ai + ho_c, rp, 1:1 + wo_c, :])


def _dot_ns(ic, ws_ref, b_ref, cout):
    out = jnp.dot(ic, ws_ref[...], preferred_element_type=jnp.float32)
    return out[:, :cout] + out[:, cout:] + b_ref[...]


def _conv_body(x_ref, w1_ref, b1_ref, w2_ref, b2_ref, w3_ref, b3_ref,
               w4_ref, b4_ref, o16_ref,
               ic2, ic3, ic4, xm2, xm3, xm4, ys3, ys4):
    b = _B

    x = x_ref[...].reshape(b * 42 * 24, 192)
    y1 = jnp.maximum(
        jnp.dot(x, w1_ref[...], preferred_element_type=jnp.float32)
        + b1_ref[...], 0.0)
    _store_pairform(y1.reshape(b, 42, 24, 64), xm2, 42, 42)

    _gather_ic(xm2, ic2, 32, 22, 24)
    y2 = jnp.maximum(
        _dot_ns(ic2[...].reshape(b * 22 * 24, 288), w2_ref, b2_ref, 64),
        0.0)
    _store_parity(y2.reshape(b, 22, 24, 64), ys3, xm3, 21, 21)

    _gather_ic(xm3, ic3, 64, 12, 16)
    y3 = jnp.maximum(
        _dot_ns(ic3[...].reshape(b * 12 * 16, 576), w3_ref, b3_ref, 128),
        0.0)
    _store_parity(y3.reshape(b, 12, 16, 128), ys4, xm4, 11, 11)

    _gather_ic(xm4, ic4, 128, 6, 8)
    y4 = jnp.maximum(
        _dot_ns(ic4[...].reshape(b * 6 * 8, 1152), w4_ref, b4_ref, 256),
        0.0)
    y4t = y4.reshape(b, 6, 8, 256)[:, :, 0:6, :].reshape(b, 36, 256)
    o16_ref[...] = y4t.astype(jnp.bfloat16)


def _conv_stack(xw, wbs, n):
    b = _B

    def _w_spec(a):
        return pl.BlockSpec(a.shape, lambda i: (0,) * a.ndim)

    return pl.pallas_call(
        _conv_body,
        out_shape=jax.ShapeDtypeStruct((n, 36, 256), jnp.bfloat16),
        grid=(n // b,),
        in_specs=([pl.BlockSpec((b, 42 * 24, 192), lambda i: (i, 0, 0))]
                  + [_w_spec(a) for a in wbs]),
        out_specs=pl.BlockSpec((b, 36, 256), lambda i: (i, 0, 0)),
        scratch_shapes=[
            pltpu.VMEM((b, 22, 24, 288), jnp.bfloat16),
            pltpu.VMEM((b, 12, 16, 576), jnp.bfloat16),
            pltpu.VMEM((b, 6, 8, 1152), jnp.bfloat16),
            pltpu.VMEM((b, 23, 2, 25, 64), jnp.bfloat16),
            pltpu.VMEM((b, 13, 2, 17, 128), jnp.bfloat16),
            pltpu.VMEM((b, 7, 2, 9, 256), jnp.bfloat16),
            pltpu.VMEM((b, 22, 24, 64), jnp.float32),
            pltpu.VMEM((b, 12, 16, 128), jnp.float32),
        ],
        compiler_params=pltpu.CompilerParams(
            dimension_semantics=("parallel",)),
    )(xw, *wbs)


def _fc_body(f_ref, w1h_ref, w1l_ref, b1_ref, w2h_ref, w2l_ref, b2_ref,
             o_ref):
    f = f_ref[...]
    h = jnp.maximum(
        jnp.dot(f, w1h_ref[...], preferred_element_type=jnp.float32)
        + jnp.dot(f, w1l_ref[...], preferred_element_type=jnp.float32)
        + b1_ref[...], 0.0).astype(jnp.bfloat16)
    o_ref[...] = (jnp.dot(h, w2h_ref[...], preferred_element_type=jnp.float32)
                  + jnp.dot(h, w2l_ref[...], preferred_element_type=jnp.float32)
                  + b2_ref[...])


def _fc_head(feat16, fc1_w, fc1_b, clf_w, clf_b):
    n, k = feat16.shape
    d, c = clf_w.shape
    bm = n // 2
    w1h, w1l = _split_bf16(fc1_w)
    w2h, w2l = _split_bf16(clf_w)
    return pl.pallas_call(
        _fc_body,
        out_shape=jax.ShapeDtypeStruct((n, c), jnp.float32),
        grid=(2,),
        in_specs=[
            pl.BlockSpec((bm, k), lambda i: (i, 0)),
            pl.BlockSpec((k, d), lambda i: (0, 0)),
            pl.BlockSpec((k, d), lambda i: (0, 0)),
            pl.BlockSpec((1, d), lambda i: (0, 0)),
            pl.BlockSpec((d, c), lambda i: (0, 0)),
            pl.BlockSpec((d, c), lambda i: (0, 0)),
            pl.BlockSpec((1, c), lambda i: (0, 0)),
        ],
        out_specs=pl.BlockSpec((bm, c), lambda i: (i, 0)),
        compiler_params=pltpu.CompilerParams(
            dimension_semantics=("parallel",)),
    )(feat16, w1h, w1l, fc1_b, w2h, w2l, clf_b)


def kernel(x, w1, b1, w2, b2, w3, b3, w4, b4, fc1_w, fc1_b, clf_w, clf_b):
    n, _, h0, w0 = x.shape
    ho1, wo1 = _conv_out(h0), _conv_out(w0)

    x_nhwc = jnp.transpose(x, (0, 2, 3, 1))
    xp = jnp.pad(x_nhwc, ((0, 0), (1, 1), (1, 1), (0, 0)))
    xs = (xp.reshape(n, ho1 + 1, 2, wo1 + 1, 2, 3)
          .transpose(0, 1, 3, 2, 4, 5)
          .reshape(n, ho1 + 1, wo1 + 1, 12))
    xsp = jnp.pad(xs, ((0, 0), (0, 0), (0, 6), (0, 0)))
    wins = [xsp[:, r:r + 42, c:c + 48, :] for r in range(2) for c in range(2)]
    xw = jnp.concatenate(wins, axis=-1).reshape(n, 42, 24, 96)
    xw = (jnp.concatenate([xw, xw], axis=-1)
          .reshape(n, 42 * 24, 192).astype(jnp.bfloat16))

    w1e = jnp.zeros((2, 2, 12, 32), jnp.float32)
    for dy in range(3):
        for dx in range(3):
            lane = 6 * (dy % 2) + 3 * (dx % 2)
            w1e = w1e.at[dy // 2, dx // 2, lane:lane + 3, :].set(w1[dy, dx])

    w1h, w1l = _split_bf16(w1e.reshape(48, 32))
    w2h, w2l = _split_bf16(w2.reshape(9 * 32, 64))
    w3h, w3l = _split_bf16(w3.reshape(9 * 64, 128))
    w4h, w4l = _split_bf16(w4.reshape(9 * 128, 256))
    z = jnp.zeros((48, 32), jnp.bfloat16)
    w1p = jnp.block([[w1h, z], [z, w1h], [w1l, z], [z, w1l]])
    b1p = jnp.concatenate([b1, b1], axis=1)
    w2s = jnp.concatenate([w2h, w2l], axis=1)
    w3s = jnp.concatenate([w3h, w3l], axis=1)
    w4s = jnp.concatenate([w4h, w4l], axis=1)
    wbs = (w1p, b1p, w2s, b2, w3s, b3, w4s, b4)

    feat16 = _conv_stack(xw, wbs, n)
    logits = _fc_head(feat16.reshape(n, 36 * 256),
                      fc1_w, fc1_b, clf_w, clf_b)
    feat_pt = (jnp.transpose(feat16, (0, 2, 1))
               .astype(jnp.float32).reshape(n, -1))
    return feat_pt, logits
